# baseline (speedup 1.0000x reference)
import functools

import jax
import jax.numpy as jnp
from jax import lax
from jax.experimental import pallas as pl
from jax.experimental.pallas import tpu as pltpu
from jax.experimental.pallas import tpu_sc as plsc

_N = 8192
_D = 128
_Q = 65536
_NC = 2
_NS = 16
_NW = _NC * _NS
_QPW = _Q // _NW
_C = 32
_NCHUNK = _QPW // _C
_L = 16

_GD = lax.GatherDimensionNumbers(
    offset_dims=(), collapsed_slice_dims=(0,), start_index_map=(0,))


def _weight_body(emb_ref, bias_ref, sw_ref, o_ref):
    o_ref[...] = (emb_ref[...] + bias_ref[...]) * sw_ref[...]


def _make_weighted(emb, bias, sw):
    return pl.pallas_call(
        _weight_body,
        out_shape=jax.ShapeDtypeStruct((_N, _D), jnp.float32),
        grid=(8,),
        in_specs=[
            pl.BlockSpec((_N // 8, _D), lambda i: (i, 0)),
            pl.BlockSpec((_N // 8, _D), lambda i: (i, 0)),
            pl.BlockSpec((1, _D), lambda i: (0, 0)),
        ],
        out_specs=pl.BlockSpec((_N // 8, _D), lambda i: (i, 0)),
    )(emb, bias, sw.reshape(1, _D))


def _sc_body(w_hbm, t_hbm, tens_hbm, curv_hbm, out_hbm,
             tens_v, curv_v, t_all,
             idx0, idx1, idx2, idx3,
             w0_v, w1_v, w2_v, w3_v,
             rowsA0, rowsA1, rowsA2, rowsA3,
             rowsB0, rowsB1, rowsB2, rowsB3,
             out_v, semA, semB):
    wid = lax.axis_index("s") * _NC + lax.axis_index("c")
    qbase = wid * _QPW
    idxs = (idx0, idx1, idx2, idx3)
    rowsA = (rowsA0, rowsA1, rowsA2, rowsA3)
    rowsB = (rowsB0, rowsB1, rowsB2, rowsB3)

    pltpu.sync_copy(tens_hbm, tens_v)
    pltpu.sync_copy(curv_hbm, curv_v)
    pltpu.sync_copy(t_hbm.at[pl.ds(qbase, _QPW)], t_all)

    def pre(k, carry):
        sl = pl.ds(k * _L, _L)
        tv = t_all[sl]
        ts = tv * float(_N - 1)
        seg = jnp.clip(ts.astype(jnp.int32), 0, _N - 2)
        tl = ts - seg.astype(jnp.float32)
        idx0[sl] = jnp.maximum(seg - 1, 0)
        idx1[sl] = seg
        idx2[sl] = seg + 1
        idx3[sl] = jnp.minimum(seg + 2, _N - 1)
        tens = plsc.load_gather(tens_v, [seg])
        sig = 1.0 / (1.0 + jnp.exp(-tens))
        c1 = plsc.load_gather(curv_v, [seg])
        c2 = plsc.load_gather(curv_v, [seg + 1])
        t2 = tl * tl
        t3 = t2 * tl
        v0 = (-0.5 * t3 + t2 - 0.5 * tl) * sig
        v1 = (1.5 * t3 - 2.5 * t2 + 1.0) * c1
        v2 = (-1.5 * t3 + 2.0 * t2 + 0.5 * tl) * c2
        v3 = (0.5 * t3 - 0.5 * t2) * sig
        rcp = 1.0 / (v0 + v1 + v2 + v3)
        w0_v[sl] = v0 * rcp
        w1_v[sl] = v1 * rcp
        w2_v[sl] = v2 * rcp
        w3_v[sl] = v3 * rcp
        return carry

    lax.fori_loop(0, _QPW // _L, pre, 0)

    def fire(c, rows, sem):
        csl = pl.ds(c * _C, _C)
        for k in range(4):
            pltpu.async_copy(w_hbm.at[idxs[k].at[csl]], rows[k], sem)

    def drain(rows, sem):
        for k in range(4):
            pltpu.make_async_copy(w_hbm.at[pl.ds(0, _C)], rows[k], sem).wait()

    def blend_store(c, rows):
        for g in range(_C // _L):
            gsl = pl.ds(c * _C + g * _L, _L)
            gw0 = w0_v[gsl]
            gw1 = w1_v[gsl]
            gw2 = w2_v[gsl]
            gw3 = w3_v[gsl]
            for i in range(_L):
                bi = jnp.full((_L, 1), i, jnp.int32)
                bw0 = lax.gather(gw0, bi, _GD, (1,),
                                 mode=lax.GatherScatterMode.PROMISE_IN_BOUNDS)
                bw1 = lax.gather(gw1, bi, _GD, (1,),
                                 mode=lax.GatherScatterMode.PROMISE_IN_BOUNDS)
                bw2 = lax.gather(gw2, bi, _GD, (1,),
                                 mode=lax.GatherScatterMode.PROMISE_IN_BOUNDS)
                bw3 = lax.gather(gw3, bi, _GD, (1,),
                                 mode=lax.GatherScatterMode.PROMISE_IN_BOUNDS)
                qi = g * _L + i
                for d in range(_D // _L):
                    dsl = pl.ds(d * _L, _L)
                    out_v[qi, dsl] = (rows[0][qi, dsl] * bw0 + rows[1][qi, dsl] * bw1
                                      + rows[2][qi, dsl] * bw2 + rows[3][qi, dsl] * bw3)
        pltpu.sync_copy(out_v, out_hbm.at[pl.ds(qbase + c * _C, _C)])

    fire(0, rowsA, semA)

    def body(i2, carry):
        ca = 2 * i2
        cb = ca + 1
        fire(cb, rowsB, semB)
        drain(rowsA, semA)
        blend_store(ca, rowsA)
        fire(jnp.minimum(ca + 2, _NCHUNK - 1), rowsA, semA)
        drain(rowsB, semB)
        blend_store(cb, rowsB)
        return carry

    lax.fori_loop(0, _NCHUNK // 2, body, 0)
    drain(rowsA, semA)


@functools.partial(
    pl.kernel,
    out_type=jax.ShapeDtypeStruct((_Q, _D), jnp.float32),
    mesh=plsc.VectorSubcoreMesh(core_axis_name="c", subcore_axis_name="s"),
    scratch_types=[
        pltpu.VMEM((_N,), jnp.float32),
        pltpu.VMEM((_N,), jnp.float32),
        pltpu.VMEM((_QPW,), jnp.float32),
        pltpu.VMEM((_QPW,), jnp.int32),
        pltpu.VMEM((_QPW,), jnp.int32),
        pltpu.VMEM((_QPW,), jnp.int32),
        pltpu.VMEM((_QPW,), jnp.int32),
        pltpu.VMEM((_QPW,), jnp.float32),
        pltpu.VMEM((_QPW,), jnp.float32),
        pltpu.VMEM((_QPW,), jnp.float32),
        pltpu.VMEM((_QPW,), jnp.float32),
        pltpu.VMEM((_C, _D), jnp.float32),
        pltpu.VMEM((_C, _D), jnp.float32),
        pltpu.VMEM((_C, _D), jnp.float32),
        pltpu.VMEM((_C, _D), jnp.float32),
        pltpu.VMEM((_C, _D), jnp.float32),
        pltpu.VMEM((_C, _D), jnp.float32),
        pltpu.VMEM((_C, _D), jnp.float32),
        pltpu.VMEM((_C, _D), jnp.float32),
        pltpu.VMEM((_C, _D), jnp.float32),
        pltpu.SemaphoreType.DMA,
        pltpu.SemaphoreType.DMA,
    ],
    compiler_params=pltpu.CompilerParams(needs_layout_passes=False),
)
def _sc_spline(w_hbm, t_hbm, tens_hbm, curv_hbm, out_hbm, *scratch):
    _sc_body(w_hbm, t_hbm, tens_hbm, curv_hbm, out_hbm, *scratch)


def kernel(word_embeddings, t_query, tension_params, semantic_weights,
           word_biases, curvature_controls):
    weighted = _make_weighted(word_embeddings, word_biases, semantic_weights)
    tens_pad = jnp.pad(tension_params, (0, 1))
    return _sc_spline(weighted, t_query, tens_pad, curvature_controls)


# speedup vs baseline: 1.0296x; 1.0296x over previous
import functools

import jax
import jax.numpy as jnp
from jax import lax
from jax.experimental import pallas as pl
from jax.experimental.pallas import tpu as pltpu
from jax.experimental.pallas import tpu_sc as plsc

_N = 8192
_D = 128
_Q = 65536
_NC = 2
_NS = 16
_NW = _NC * _NS
_QPW = _Q // _NW
_C = 32
_NCHUNK = _QPW // _C
_L = 16

_GD = lax.GatherDimensionNumbers(
    offset_dims=(), collapsed_slice_dims=(0,), start_index_map=(0,))


def _weight_body(emb_ref, bias_ref, sw_ref, o_ref):
    o_ref[...] = (emb_ref[...] + bias_ref[...]) * sw_ref[...]


def _make_weighted(emb, bias, sw):
    return pl.pallas_call(
        _weight_body,
        out_shape=jax.ShapeDtypeStruct((_N, _D), jnp.float32),
        grid=(8,),
        in_specs=[
            pl.BlockSpec((_N // 8, _D), lambda i: (i, 0)),
            pl.BlockSpec((_N // 8, _D), lambda i: (i, 0)),
            pl.BlockSpec((1, _D), lambda i: (0, 0)),
        ],
        out_specs=pl.BlockSpec((_N // 8, _D), lambda i: (i, 0)),
    )(emb, bias, sw.reshape(1, _D))


def _sc_body(w_hbm, t_hbm, tens_hbm, curv_hbm, out_hbm,
             tens_v, curv_v, t_all,
             idxi,
             w0_v, w1_v, w2_v, w3_v,
             rowsA, rowsB, outA, outB,
             semA, semB, semOA, semOB):
    wid = lax.axis_index("s") * _NC + lax.axis_index("c")
    qbase = wid * _QPW

    pltpu.sync_copy(tens_hbm, tens_v)
    pltpu.sync_copy(curv_hbm, curv_v)
    pltpu.sync_copy(t_hbm.at[pl.ds(qbase, _QPW)], t_all)

    lanes4 = lax.iota(jnp.int32, _L) * 4

    def pre(k, carry):
        sl = pl.ds(k * _L, _L)
        tv = t_all[sl]
        ts = tv * float(_N - 1)
        seg = jnp.clip(ts.astype(jnp.int32), 0, _N - 2)
        tl = ts - seg.astype(jnp.float32)
        pos = k * (4 * _L) + lanes4
        plsc.store_scatter(idxi, [pos], jnp.maximum(seg - 1, 0))
        plsc.store_scatter(idxi, [pos + 1], seg)
        plsc.store_scatter(idxi, [pos + 2], seg + 1)
        plsc.store_scatter(idxi, [pos + 3], jnp.minimum(seg + 2, _N - 1))
        tens = plsc.load_gather(tens_v, [seg])
        sig = 1.0 / (1.0 + jnp.exp(-tens))
        c1 = plsc.load_gather(curv_v, [seg])
        c2 = plsc.load_gather(curv_v, [seg + 1])
        t2 = tl * tl
        t3 = t2 * tl
        v0 = (-0.5 * t3 + t2 - 0.5 * tl) * sig
        v1 = (1.5 * t3 - 2.5 * t2 + 1.0) * c1
        v2 = (-1.5 * t3 + 2.0 * t2 + 0.5 * tl) * c2
        v3 = (0.5 * t3 - 0.5 * t2) * sig
        rcp = 1.0 / (v0 + v1 + v2 + v3)
        w0_v[sl] = v0 * rcp
        w1_v[sl] = v1 * rcp
        w2_v[sl] = v2 * rcp
        w3_v[sl] = v3 * rcp
        return carry

    lax.fori_loop(0, _QPW // _L, pre, 0)

    def fire(c, rows, sem):
        pltpu.async_copy(w_hbm.at[idxi.at[pl.ds(c * (4 * _C), 4 * _C)]],
                         rows, sem)

    def drain(rows, sem):
        pltpu.make_async_copy(w_hbm.at[pl.ds(0, 4 * _C)], rows, sem).wait()

    def blend(c, rows, out_v):
        for g in range(_C // _L):
            gsl = pl.ds(c * _C + g * _L, _L)
            gw0 = w0_v[gsl]
            gw1 = w1_v[gsl]
            gw2 = w2_v[gsl]
            gw3 = w3_v[gsl]
            for i in range(_L):
                bi = jnp.full((_L, 1), i, jnp.int32)
                bw0 = lax.gather(gw0, bi, _GD, (1,),
                                 mode=lax.GatherScatterMode.PROMISE_IN_BOUNDS)
                bw1 = lax.gather(gw1, bi, _GD, (1,),
                                 mode=lax.GatherScatterMode.PROMISE_IN_BOUNDS)
                bw2 = lax.gather(gw2, bi, _GD, (1,),
                                 mode=lax.GatherScatterMode.PROMISE_IN_BOUNDS)
                bw3 = lax.gather(gw3, bi, _GD, (1,),
                                 mode=lax.GatherScatterMode.PROMISE_IN_BOUNDS)
                qi = g * _L + i
                for d in range(_D // _L):
                    dsl = pl.ds(d * _L, _L)
                    out_v[qi, dsl] = (rows[4 * qi, dsl] * bw0
                                      + rows[4 * qi + 1, dsl] * bw1
                                      + rows[4 * qi + 2, dsl] * bw2
                                      + rows[4 * qi + 3, dsl] * bw3)

    def store(c, out_v, sem):
        pltpu.async_copy(out_v, out_hbm.at[pl.ds(qbase + c * _C, _C)], sem)

    def drain_store(out_v, sem):
        pltpu.make_async_copy(out_v, out_hbm.at[pl.ds(qbase, _C)], sem).wait()

    fire(0, rowsA, semA)

    def body(i2, carry):
        ca = 2 * i2
        cb = ca + 1
        fire(cb, rowsB, semB)
        drain(rowsA, semA)

        @pl.when(i2 > 0)
        def _():
            drain_store(outA, semOA)

        blend(ca, rowsA, outA)
        store(ca, outA, semOA)
        fire(jnp.minimum(ca + 2, _NCHUNK - 1), rowsA, semA)
        drain(rowsB, semB)

        @pl.when(i2 > 0)
        def _():
            drain_store(outB, semOB)

        blend(cb, rowsB, outB)
        store(cb, outB, semOB)
        return carry

    lax.fori_loop(0, _NCHUNK // 2, body, 0)
    drain(rowsA, semA)
    drain_store(outA, semOA)
    drain_store(outB, semOB)


@functools.partial(
    pl.kernel,
    out_type=jax.ShapeDtypeStruct((_Q, _D), jnp.float32),
    mesh=plsc.VectorSubcoreMesh(core_axis_name="c", subcore_axis_name="s"),
    scratch_types=[
        pltpu.VMEM((_N,), jnp.float32),
        pltpu.VMEM((_N,), jnp.float32),
        pltpu.VMEM((_QPW,), jnp.float32),
        pltpu.VMEM((4 * _QPW,), jnp.int32),
        pltpu.VMEM((_QPW,), jnp.float32),
        pltpu.VMEM((_QPW,), jnp.float32),
        pltpu.VMEM((_QPW,), jnp.float32),
        pltpu.VMEM((_QPW,), jnp.float32),
        pltpu.VMEM((4 * _C, _D), jnp.float32),
        pltpu.VMEM((4 * _C, _D), jnp.float32),
        pltpu.VMEM((_C, _D), jnp.float32),
        pltpu.VMEM((_C, _D), jnp.float32),
        pltpu.SemaphoreType.DMA,
        pltpu.SemaphoreType.DMA,
        pltpu.SemaphoreType.DMA,
        pltpu.SemaphoreType.DMA,
    ],
    compiler_params=pltpu.CompilerParams(needs_layout_passes=False),
)
def _sc_spline(w_hbm, t_hbm, tens_hbm, curv_hbm, out_hbm, *scratch):
    _sc_body(w_hbm, t_hbm, tens_hbm, curv_hbm, out_hbm, *scratch)


def kernel(word_embeddings, t_query, tension_params, semantic_weights,
           word_biases, curvature_controls):
    weighted = _make_weighted(word_embeddings, word_biases, semantic_weights)
    tens_pad = jnp.pad(tension_params, (0, 1))
    return _sc_spline(weighted, t_query, tens_pad, curvature_controls)
